# baseline (speedup 1.0000x reference)
import functools

import jax
import jax.numpy as jnp
from jax import lax
from jax.experimental import pallas as pl
from jax.experimental.pallas import tpu as pltpu
from jax.experimental.pallas import tpu_sc as plsc

N = 50000
E = 800000
G = 128
HID = 16
HEADS = 4

NB = 1000
NBLK = N // NB
PB = 2000
PBLK = N // PB

EB = 128
EBLK = E // EB
NPT = 3128
NPT_LAST = N - 15 * NPT

_F32 = jnp.float32


def _gelu(v):
    return v * 0.5 * (1.0 + lax.erf(v * 0.7071067811865476))


def _ln(v, g, b):
    m = jnp.mean(v, axis=-1, keepdims=True)
    var = jnp.mean((v - m) ** 2, axis=-1, keepdims=True)
    return (v - m) / jnp.sqrt(var + 1e-5) * g + b


def _head_split(xl):
    h = xl.shape[-1] // HID
    return jnp.transpose(xl.reshape(xl.shape[0], h, HID), (1, 0, 2))


def _in_body(x_ref, wi_ref, bi_ref, wl_ref, bl_ref, wr_ref, br_ref,
             h_ref, xl_ref, xr_ref):
    h = _gelu(jnp.dot(x_ref[...], wi_ref[...],
                      preferred_element_type=_F32) + bi_ref[...])
    h_ref[...] = h
    xl_ref[...] = jnp.dot(h, wl_ref[...],
                          preferred_element_type=_F32) + bl_ref[...]
    xr_ref[...] = jnp.dot(h, wr_ref[...],
                          preferred_element_type=_F32) + br_ref[...]


def _tc_input(x, wi, bi, wl, bl, wr, br):
    return pl.pallas_call(
        _in_body,
        grid=(NBLK,),
        in_specs=[
            pl.BlockSpec((NB, 6), lambda i: (i, 0)),
            pl.BlockSpec((6, 64), lambda i: (0, 0)),
            pl.BlockSpec((64,), lambda i: (0,)),
            pl.BlockSpec((64, 64), lambda i: (0, 0)),
            pl.BlockSpec((64,), lambda i: (0,)),
            pl.BlockSpec((64, 64), lambda i: (0, 0)),
            pl.BlockSpec((64,), lambda i: (0,)),
        ],
        out_specs=[
            pl.BlockSpec((NB, 64), lambda i: (i, 0)),
            pl.BlockSpec((NB, 64), lambda i: (i, 0)),
            pl.BlockSpec((NB, 64), lambda i: (i, 0)),
        ],
        out_shape=[
            jax.ShapeDtypeStruct((N, 64), _F32),
            jax.ShapeDtypeStruct((N, 64), _F32),
            jax.ShapeDtypeStruct((N, 64), _F32),
        ],
    )(x, wi, bi, wl, bl, wr, br)


def _mid_body(nh, acc_ref, hp_ref, bias_ref, g_ref, b_ref,
              wl_ref, bl_ref, wr_ref, br_ref, h_ref, xl_ref, xr_ref):
    num = acc_ref[:, :, 0:HID]
    den = acc_ref[:, :, HID:HID + 1] + 1e-16
    o = jnp.transpose(num / den, (1, 0, 2)).reshape(-1, 64)
    o = _gelu(_ln(o + bias_ref[...], g_ref[...], b_ref[...]))
    h = hp_ref[...] + o
    h_ref[...] = h
    xl_ref[...] = jnp.dot(h, wl_ref[...],
                          preferred_element_type=_F32) + bl_ref[...]
    xr_ref[...] = jnp.dot(h, wr_ref[...],
                          preferred_element_type=_F32) + br_ref[...]


def _tc_mid(accs, h_prev, bias, ln_g, ln_b, wl, bl, wr, br, next_heads):
    nh = next_heads
    return pl.pallas_call(
        functools.partial(_mid_body, nh),
        grid=(NBLK,),
        in_specs=[
            pl.BlockSpec((4, NB, 32), lambda i: (0, i, 0)),
            pl.BlockSpec((NB, 64), lambda i: (i, 0)),
            pl.BlockSpec((64,), lambda i: (0,)),
            pl.BlockSpec((64,), lambda i: (0,)),
            pl.BlockSpec((64,), lambda i: (0,)),
            pl.BlockSpec((64, nh * HID), lambda i: (0, 0)),
            pl.BlockSpec((nh * HID,), lambda i: (0,)),
            pl.BlockSpec((64, nh * HID), lambda i: (0, 0)),
            pl.BlockSpec((nh * HID,), lambda i: (0,)),
        ],
        out_specs=[
            pl.BlockSpec((NB, 64), lambda i: (i, 0)),
            pl.BlockSpec((NB, nh * HID), lambda i: (i, 0)),
            pl.BlockSpec((NB, nh * HID), lambda i: (i, 0)),
        ],
        out_shape=[
            jax.ShapeDtypeStruct((N, 64), _F32),
            jax.ShapeDtypeStruct((N, nh * HID), _F32),
            jax.ShapeDtypeStruct((N, nh * HID), _F32),
        ],
    )(accs, h_prev, bias, ln_g, ln_b, wl, bl, wr, br)


def _fin_body(acc_ref, bias_ref, g_ref, b_ref, h_ref):
    num = acc_ref[0, :, 0:HID] + acc_ref[1, :, 0:HID]
    den = acc_ref[0, :, HID:HID + 1] + acc_ref[1, :, HID:HID + 1] + 1e-16
    o = num / den + bias_ref[...]
    h_ref[...] = _gelu(_ln(o, g_ref[...], b_ref[...]))


def _tc_fin(accs, bias, ln_g, ln_b):
    return pl.pallas_call(
        _fin_body,
        grid=(NBLK,),
        in_specs=[
            pl.BlockSpec((2, NB, 32), lambda i: (0, i, 0)),
            pl.BlockSpec((HID,), lambda i: (0,)),
            pl.BlockSpec((HID,), lambda i: (0,)),
            pl.BlockSpec((HID,), lambda i: (0,)),
        ],
        out_specs=pl.BlockSpec((NB, HID), lambda i: (i, 0)),
        out_shape=jax.ShapeDtypeStruct((N, HID), _F32),
    )(accs, bias, ln_g, ln_b)


def _pool_body(h_ref, b_ref, sum_ref, cnt_ref, max_ref):
    i = pl.program_id(0)

    @pl.when(i == 0)
    def _():
        sum_ref[...] = jnp.zeros((G, HID), _F32)
        cnt_ref[...] = jnp.zeros((G, HID), _F32)
        max_ref[...] = jnp.full((G, HID), -3.4e38, _F32)

    b = b_ref[0, 0, :]
    h = h_ref[...]
    bmin = jnp.min(b)
    bmax = jnp.max(b)
    gids = lax.broadcasted_iota(jnp.int32, (PB, G), 1)
    mask = (b[:, None] == gids).astype(_F32)
    sum_ref[...] += lax.dot_general(mask, h, (((0,), (0,)), ((), ())),
                                    preferred_element_type=_F32)
    cnt_ref[...] += jnp.broadcast_to(jnp.sum(mask, axis=0)[:, None], (G, HID))
    for gg in range(G // 8):
        @pl.when(jnp.logical_and(bmin <= gg * 8 + 7, bmax >= gg * 8))
        def _(gg=gg):
            rows = []
            for g in range(8):
                sel = jnp.where(b[:, None] == gg * 8 + g, h, -3.4e38)
                rows.append(jnp.max(sel, axis=0))
            blk = jnp.stack(rows, axis=0)
            cur = max_ref[gg * 8:(gg + 1) * 8, :]
            max_ref[gg * 8:(gg + 1) * 8, :] = jnp.maximum(cur, blk)


def _tc_pool(h_fin, batch_r):
    return pl.pallas_call(
        _pool_body,
        grid=(PBLK,),
        in_specs=[
            pl.BlockSpec((PB, HID), lambda i: (i, 0)),
            pl.BlockSpec((1, 1, PB), lambda i: (i, 0, 0)),
        ],
        out_specs=[
            pl.BlockSpec((G, HID), lambda i: (0, 0)),
            pl.BlockSpec((G, HID), lambda i: (0, 0)),
            pl.BlockSpec((G, HID), lambda i: (0, 0)),
        ],
        out_shape=[
            jax.ShapeDtypeStruct((G, HID), _F32),
            jax.ShapeDtypeStruct((G, HID), _F32),
            jax.ShapeDtypeStruct((G, HID), _F32),
        ],
    )(h_fin, batch_r)


def _mlp_body(sum_ref, cnt_ref, max_ref, w1, b1, g1, bb1, w2, b2, g2, bb2,
              w3, b3, out_ref):
    cnt = cnt_ref[...]
    mean = sum_ref[...] / jnp.maximum(cnt, 1.0)
    mx = jnp.where(cnt > 0.0, max_ref[...], 0.0)
    g = jnp.concatenate([mean, mx], axis=-1)
    g = _gelu(_ln(jnp.dot(g, w1[...], preferred_element_type=_F32) + b1[...],
                  g1[...], bb1[...]))
    g = _gelu(_ln(jnp.dot(g, w2[...], preferred_element_type=_F32) + b2[...],
                  g2[...], bb2[...]))
    out_ref[...] = jnp.dot(g, w3[...], preferred_element_type=_F32) + b3[...]


def _tc_mlp(sums, cnts, maxs, p):
    args = (sums, cnts, maxs, p['fc1_W'], p['fc1_b'], p['ln1_g'], p['ln1_b'],
            p['fc2_W'], p['fc2_b'], p['ln2_g'], p['ln2_b'], p['fc3_W'], p['fc3_b'])
    return pl.pallas_call(
        _mlp_body,
        grid=(1,),
        in_specs=[pl.BlockSpec(a.shape, functools.partial(
                      lambda nd, i: (0,) * nd, len(a.shape)))
                  for a in args],
        out_specs=pl.BlockSpec((G, 2), lambda i: (0, 0)),
        out_shape=jax.ShapeDtypeStruct((G, 2), _F32),
    )(*args)


def _exp_sc(x):
    t = x * 1.4426950408889634
    k = (t + jnp.where(t >= 0, 0.5, -0.5)).astype(jnp.int32)
    kf = k.astype(_F32)
    r = (x - kf * 0.693359375) - kf * (-2.1219444005469058e-4)
    p = 2.4801587301587302e-05
    for c in (1.984126984126984e-04, 1.3888888888888889e-03,
              8.333333333333333e-03, 4.1666666666666664e-02,
              1.6666666666666666e-01, 0.5, 1.0, 1.0):
        p = p * r + c
    s = lax.bitcast_convert_type(jnp.left_shift(k + 127, 23), _F32)
    return p * s


def _sc_edge_body(H, xl_hbm, xr_hbm, src_hbm, dst_hbm, ea_hbm, wab_hbm,
                  zeros_hbm, accs_hbm, acc_sp, isrc, idst, ea_v, xlg, xrg,
                  orow, p_flat, wab_v, sem1, sem2):
    cid = lax.axis_index("c")
    sid = lax.axis_index("s")
    iota = lax.iota(jnp.int32, 16)

    if H == 4:
        nblk = jnp.where(sid < 10, 391, 390)
        blk0 = sid * 390 + jnp.minimum(sid, 10)
        passes = 2
    else:
        wid = cid * 16 + sid
        nblk = jnp.where(wid < 10, 196, 195)
        blk0 = wid * 195 + jnp.minimum(wid, 10)
        passes = 1

    for pi in range(passes):
        head = 2 * cid + pi if H == 4 else 0
        out_idx = head if H == 4 else cid

        colbase = head * HID
        @pl.when(sid < 15)
        def _():
            pltpu.sync_copy(zeros_hbm, acc_sp.at[pl.ds(sid * NPT, NPT)])

        @pl.when(sid == 15)
        def _():
            pltpu.sync_copy(zeros_hbm.at[pl.ds(0, NPT_LAST)],
                            acc_sp.at[pl.ds(15 * NPT, NPT_LAST)])

        pltpu.sync_copy(wab_hbm.at[head], wab_v)
        plsc.subcore_barrier()

        def block_body(bi, carry):
            e0 = (blk0 + bi) * EB
            pltpu.sync_copy(src_hbm.at[pl.ds(e0, EB)], isrc)
            pltpu.sync_copy(dst_hbm.at[pl.ds(e0, EB)], idst)
            pltpu.sync_copy(ea_hbm.at[pl.ds(e0, EB)], ea_v)
            c1 = pltpu.async_copy(xl_hbm.at[isrc], xlg, sem1)
            c2 = pltpu.async_copy(xr_hbm.at[idst], xrg, sem2)
            c1.wait()
            c2.wait()

            def group_tile(gt, carry2):
                rowv = [gt * 64 + j * 16 + iota for j in range(4)]
                logit = [None] * 4
                for c in range(16):
                    cc = jnp.full((16,), c, jnp.int32)
                    cch = cc + colbase
                    ws = [plsc.load_gather(
                              wab_v, [jnp.full((16,), r, jnp.int32), cc])
                          for r in range(4, 6)]
                    for j in range(4):
                        xlc = plsc.load_gather(xlg, [rowv[j], cch])
                        xrc = plsc.load_gather(xrg, [rowv[j], cch])
                        eec = plsc.load_gather(ea_v, [rowv[j], cch])
                        u = xlc + xrc + eec
                        t = ws[0] * u + ws[1] * jnp.abs(u)
                        logit[j] = t if c == 0 else logit[j] + t
                pv = [_exp_sc(logit[j]) for j in range(4)]
                for j in range(4):
                    p_flat[pl.ds((gt * 4 + j) * 16, 16)] = pv[j]
                return carry2

            lax.fori_loop(0, EB // 64, group_tile, 0)

            def row_build(eb, carry2):
                for kk in range(4):
                    e = eb * 4 + kk
                    erow = jnp.full((16,), e, jnp.int32)
                    xlr = plsc.load_gather(xlg, [erow, iota + colbase])
                    psp = plsc.load_gather(p_flat, [erow])
                    orow[e, 0:HID] = psp * xlr
                    orow[e, HID:2 * HID] = psp
                return carry2

            lax.fori_loop(0, EB // 4, row_build, 0)
            pltpu.sync_copy(orow, acc_sp.at[idst], add=True)
            return carry

        lax.fori_loop(0, nblk, block_body, 0)
        plsc.subcore_barrier()

        @pl.when(sid < 15)
        def _():
            pltpu.sync_copy(acc_sp.at[pl.ds(sid * NPT, NPT)],
                            accs_hbm.at[out_idx].at[pl.ds(sid * NPT, NPT)])

        @pl.when(sid == 15)
        def _():
            pltpu.sync_copy(acc_sp.at[pl.ds(15 * NPT, NPT_LAST)],
                            accs_hbm.at[out_idx].at[pl.ds(15 * NPT, NPT_LAST)])

        plsc.subcore_barrier()


def _sc_edge(xl_h, xr_h, src, dst, ea, wab, zeros_h, H):
    nout = 4 if H == 4 else 2
    mesh = plsc.VectorSubcoreMesh(core_axis_name="c", subcore_axis_name="s",
                                  num_cores=2, num_subcores=16)
    return pl.kernel(
        functools.partial(_sc_edge_body, H),
        out_type=jax.ShapeDtypeStruct((nout, N, 32), _F32),
        mesh=mesh,
        compiler_params=pltpu.CompilerParams(needs_layout_passes=False,
                                             use_tc_tiling_on_sc=False),
        scratch_types=[
            pltpu.VMEM_SHARED((N, 32), _F32),
            pltpu.VMEM((EB,), jnp.int32),
            pltpu.VMEM((EB,), jnp.int32),
            pltpu.VMEM((EB, 64), _F32),
            pltpu.VMEM((EB, 64), _F32),
            pltpu.VMEM((EB, 64), _F32),
            pltpu.VMEM((EB, 32), _F32),
            pltpu.VMEM((EB,), _F32),
            pltpu.VMEM((6, HID), _F32),
            pltpu.SemaphoreType.DMA,
            pltpu.SemaphoreType.DMA,
        ],
    )(xl_h, xr_h, src, dst, ea, wab, zeros_h)


def _make_wab(we, att):
    h = att.shape[1]
    wes = jnp.transpose(we.reshape(4, h, HID), (1, 0, 2))
    a = 0.6 * att[0][:, None, :]
    b = 0.4 * att[0][:, None, :]
    return jnp.concatenate([wes, a, b], axis=1)


def kernel(x, edge_index, edge_attr, batch, params):
    src = edge_index[0]
    dst = edge_index[1]
    zeros_h = jnp.zeros((NPT, 32), _F32)

    h, xl, xr = _tc_input(x, params['in_W'], params['in_b'],
                          params['l0_Wl'], params['l0_bl'],
                          params['l0_Wr'], params['l0_br'])
    for i in range(2):
        wab = _make_wab(params['l%d_We' % i], params['l%d_att' % i])
        ee = edge_attr @ params['l%d_We' % i]
        accs = _sc_edge(xl, xr, src, dst, ee, wab, zeros_h, 4)
        nh = 4 if i == 0 else 1
        h, xl, xr = _tc_mid(accs, h,
                            params['l%d_bias' % i], params['l%d_ln_g' % i],
                            params['l%d_ln_b' % i],
                            params['l%d_Wl' % (i + 1)], params['l%d_bl' % (i + 1)],
                            params['l%d_Wr' % (i + 1)], params['l%d_br' % (i + 1)],
                            nh)
        if nh == 1:
            xl = jnp.pad(xl, ((0, 0), (0, 64 - HID)))
            xr = jnp.pad(xr, ((0, 0), (0, 64 - HID)))

    wab2 = _make_wab(params['l2_We'], params['l2_att'])
    ee2 = jnp.pad(edge_attr @ params['l2_We'], ((0, 0), (0, 64 - HID)))
    accs2 = _sc_edge(xl, xr, src, dst, ee2, wab2, zeros_h, 1)
    h_fin = _tc_fin(accs2, params['l2_bias'], params['l2_ln_g'], params['l2_ln_b'])

    sums, cnts, maxs = _tc_pool(h_fin, batch.reshape(PBLK, 1, PB))
    return _tc_mlp(sums, cnts, maxs, params)
